# baseline (speedup 1.0000x reference)
import functools

import jax
import jax.numpy as jnp
from jax import lax
from jax.experimental import pallas as pl
from jax.experimental.pallas import tpu as pltpu
from jax.experimental.pallas import tpu_sc as plsc

N = 10000
E = 320000
D = 128
H = 128
C = 40
CP = 128

NC = 2
NS = 16
L = 16
NW = NC * NS

NPAD = 10240
RPT = NPAD // NS
ETOT = E + N
K = 112
NCHUNK = 96
EW = NCHUNK * K
EPAD = EW * NW

@functools.cache
def _mesh():
    return plsc.VectorSubcoreMesh(
        core_axis_name="c", subcore_axis_name="s",
        num_cores=NC, num_subcores=NS)


_SC_PARAMS = pltpu.CompilerParams(needs_layout_passes=False)


def _sc_pre_body(ed_in, ed_out, acc, sbuf, tbuf, dinvv, ebw,
                 degsh, dinvsh):
    cid = lax.axis_index("c")
    sid = lax.axis_index("s")
    wid = cid * NS + sid

    zero = jnp.zeros((L,), jnp.float32)

    @pl.loop(0, NPAD // L)
    def _zero(i):
        acc[pl.ds(i * L, L)] = zero

    for woff in (0, NS):
        pltpu.sync_copy(ed_in.at[sid + woff], ebw)

        @pl.loop(0, NCHUNK)
        def _chunk(t):
            for j in range(K // L):
                d16 = ebw[t * 3 + 1, pl.ds(j * L, L)]
                w16 = plsc.bitcast(ebw[t * 3 + 2, pl.ds(j * L, L)],
                                   jnp.float32)
                plsc.addupdate_scatter(acc, [d16], w16)

    for s in range(NS):
        pltpu.sync_copy(acc.at[pl.ds(s * RPT, RPT)], degsh.at[s, sid])
    plsc.subcore_barrier()

    pltpu.sync_copy(degsh.at[sid], tbuf)

    @pl.loop(0, RPT // L)
    def _red(i):
        s = tbuf[0, pl.ds(i * L, L)]
        for w in range(1, NS):
            s = s + tbuf[w, pl.ds(i * L, L)]
        x = jnp.maximum(s, 1.0)
        yi = jnp.int32(0x5F3759DF) - (plsc.bitcast(x, jnp.int32) >> 1)
        y = plsc.bitcast(yi, jnp.float32)
        h = 0.5 * x
        y = y * (1.5 - h * y * y)
        y = y * (1.5 - h * y * y)
        y = y * (1.5 - h * y * y)
        sbuf[pl.ds(i * L, L)] = y

    pltpu.sync_copy(sbuf, dinvsh.at[sid])
    plsc.subcore_barrier()
    for s in range(NS):
        pltpu.sync_copy(dinvsh.at[s], dinvv.at[pl.ds(s * RPT, RPT)])

    pltpu.sync_copy(ed_in.at[wid], ebw)

    @pl.loop(0, NCHUNK)
    def _cchunk(t):
        for j in range(K // L):
            s16 = ebw[t * 3, pl.ds(j * L, L)]
            d16 = ebw[t * 3 + 1, pl.ds(j * L, L)]
            w16 = plsc.bitcast(ebw[t * 3 + 2, pl.ds(j * L, L)], jnp.float32)
            c16 = w16 * plsc.load_gather(dinvv, [s16]) \
                      * plsc.load_gather(dinvv, [d16])
            ebw[t * 3 + 2, pl.ds(j * L, L)] = plsc.bitcast(c16, jnp.int32)

    pltpu.sync_copy(ebw, ed_out.at[wid])


@functools.cache
def _sc_pre():
    return pl.kernel(
        _sc_pre_body,
        out_type=jax.ShapeDtypeStruct((NW, NCHUNK * 3, K), jnp.int32),
        mesh=_mesh(),
        compiler_params=_SC_PARAMS,
        scratch_types=[
            pltpu.VMEM((NPAD,), jnp.float32),
            pltpu.VMEM((RPT,), jnp.float32),
            pltpu.VMEM((NS, RPT), jnp.float32),
            pltpu.VMEM((NPAD,), jnp.float32),
            pltpu.VMEM((NCHUNK * 3, K), jnp.int32),
            pltpu.VMEM_SHARED((NS, NS, RPT), jnp.float32),
            pltpu.VMEM_SHARED((NS, RPT), jnp.float32),
        ],
    )


def _make_sc_agg(hp):
    def body(xw_hbm, ed_hbm, out_hbm,
             accs, eb0, eb1, eb2, eb3, eb4, eb5, rw0, rw1, rw2,
             es0, es1, es2, es3, es4, es5, gs0, gs1, gs2, ss0, ss1, ss2):
        cid = lax.axis_index("c")
        sid = lax.axis_index("s")
        wid = cid * NS + sid

        eb = (eb0, eb1, eb2, eb3, eb4, eb5)
        es = (es0, es1, es2, es3, es4, es5)
        rw = (rw0, rw1, rw2)
        gs = (gs0, gs1, gs2)
        ss = (ss0, ss1, ss2)

        pltpu.sync_copy(ed_hbm.at[wid, 0], eb0)
        pltpu.sync_copy(ed_hbm.at[wid, 1], eb1)
        pltpu.async_copy(ed_hbm.at[wid, 2], eb2, es2)
        pltpu.async_copy(xw_hbm.at[eb0.at[0]], rw0, gs0)
        pltpu.async_copy(xw_hbm.at[eb1.at[0]], rw1, gs1)

        zero = jnp.zeros((L,), jnp.float32)

        @pl.loop(0, K)
        def _z0(r):
            for j in range(hp // L):
                rw2[r, pl.ds(j * L, L)] = zero

        nfull = RPT // K
        rem = RPT - nfull * K
        for i in range(nfull):
            pltpu.sync_copy(rw2, accs.at[pl.ds(sid * RPT + i * K, K)])
        if rem:
            pltpu.sync_copy(rw2.at[pl.ds(0, rem)],
                            accs.at[pl.ds(sid * RPT + nfull * K, rem)])

        plsc.subcore_barrier()

        @pl.loop(0, NCHUNK // 6)
        def _grp(g):
            for b in range(6):
                u = g * 6 + b
                b3 = b % 3
                bp3 = (b + 2) % 3
                bn3 = (b + 2) % 3

                @pl.when(u + 3 < NCHUNK)
                def _():
                    pltpu.async_copy(ed_hbm.at[wid, u + 3],
                                     eb[(b + 3) % 6], es[(b + 3) % 6])

                pltpu.make_async_copy(
                    xw_hbm.at[eb[b].at[0]], rw[b3], gs[b3]).wait()

                cref = eb[b].at[2]

                @pl.loop(0, K, unroll=8)
                def _row(r):
                    cr = plsc.bitcast(
                        plsc.load_gather(cref, [jnp.full((L,), r, jnp.int32)]),
                        jnp.float32)
                    for j in range(hp // L):
                        rw[b3][r, pl.ds(j * L, L)] = \
                            rw[b3][r, pl.ds(j * L, L)] * cr

                @pl.when(u >= 1)
                def _():
                    pltpu.make_async_copy(
                        rw[bp3], accs.at[eb[(b + 5) % 6].at[1]],
                        ss[bp3]).wait()

                pltpu.async_copy(rw[b3], accs.at[eb[b].at[1]], ss[b3],
                                 add=True)

                @pl.when(u + 2 < NCHUNK)
                def _():
                    pltpu.make_async_copy(ed_hbm.at[wid, u + 2],
                                          eb[(b + 2) % 6],
                                          es[(b + 2) % 6]).wait()
                    pltpu.async_copy(xw_hbm.at[eb[(b + 2) % 6].at[0]],
                                     rw[bn3], gs[bn3])

        ul = NCHUNK - 1
        pltpu.make_async_copy(rw[ul % 3], accs.at[eb[ul % 6].at[1]],
                              ss[ul % 3]).wait()
        plsc.subcore_barrier()
        pltpu.sync_copy(accs.at[pl.ds(sid * RPT, RPT)],
                        out_hbm.at[cid, pl.ds(sid * RPT, RPT)])

    return pl.kernel(
        body,
        out_type=jax.ShapeDtypeStruct((NC, NPAD, hp), jnp.float32),
        mesh=_mesh(),
        compiler_params=_SC_PARAMS,
        scratch_types=[
            pltpu.VMEM_SHARED((NPAD, hp), jnp.float32),
            pltpu.VMEM((3, K), jnp.int32),
            pltpu.VMEM((3, K), jnp.int32),
            pltpu.VMEM((3, K), jnp.int32),
            pltpu.VMEM((3, K), jnp.int32),
            pltpu.VMEM((3, K), jnp.int32),
            pltpu.VMEM((3, K), jnp.int32),
            pltpu.VMEM((K, hp), jnp.float32),
            pltpu.VMEM((K, hp), jnp.float32),
            pltpu.VMEM((K, hp), jnp.float32),
            pltpu.SemaphoreType.DMA,
            pltpu.SemaphoreType.DMA,
            pltpu.SemaphoreType.DMA,
            pltpu.SemaphoreType.DMA,
            pltpu.SemaphoreType.DMA,
            pltpu.SemaphoreType.DMA,
            pltpu.SemaphoreType.DMA,
            pltpu.SemaphoreType.DMA,
            pltpu.SemaphoreType.DMA,
            pltpu.SemaphoreType.DMA,
            pltpu.SemaphoreType.DMA,
            pltpu.SemaphoreType.DMA,
        ],
    )


_sc_agg = functools.cache(_make_sc_agg)


def _tc_mm_body(x_ref, w_ref, o_ref):
    o_ref[...] = jnp.dot(x_ref[...], w_ref[...],
                         preferred_element_type=jnp.float32)


def _tc_mm(x, w):
    return pl.pallas_call(
        _tc_mm_body,
        out_shape=jax.ShapeDtypeStruct((x.shape[0], w.shape[1]), jnp.float32),
    )(x, w)


def _tc_bn_mm_body(acc_ref, g_ref, be_ref, w_ref, o_ref):
    a = acc_ref[0] + acc_ref[1]
    mean = jnp.sum(a, axis=0, keepdims=True) * (1.0 / N)
    dev = a - mean
    rmask = lax.broadcasted_iota(jnp.int32, (NPAD, 1), 0) < N
    devm = jnp.where(rmask, dev, 0.0)
    var = jnp.sum(devm * devm, axis=0, keepdims=True) * (1.0 / N)
    h = dev * lax.rsqrt(var + 1e-5) * g_ref[...] + be_ref[...]
    h = jnp.maximum(h, 0.0)
    o_ref[...] = jnp.dot(h, w_ref[...], preferred_element_type=jnp.float32)


def _tc_bn_mm(accs, g, be, w):
    return pl.pallas_call(
        _tc_bn_mm_body,
        out_shape=jax.ShapeDtypeStruct((NPAD, w.shape[1]), jnp.float32),
    )(accs, g.reshape(1, -1), be.reshape(1, -1), w)


def _tc_final_body(acc_ref, b_ref, o_ref):
    z = acc_ref[0] + acc_ref[1] + b_ref[...]
    cmask = lax.broadcasted_iota(jnp.int32, (1, CP), 1) < C
    z = jnp.where(cmask, z, -1e30)
    m = jnp.max(z, axis=1, keepdims=True)
    e = jnp.where(cmask, jnp.exp(z - m), 0.0)
    s = jnp.sum(e, axis=1, keepdims=True)
    out = z - m - jnp.log(s)
    o_ref[...] = out[:N, :C]


def _tc_final(accs, b3p):
    return pl.pallas_call(
        _tc_final_body,
        out_shape=jax.ShapeDtypeStruct((N, C), jnp.float32),
    )(accs, b3p.reshape(1, -1))


def kernel(x, edge_index, edge_weight, W1, b1, g1, be1,
           W2, b2, g2, be2, W3, b3):
    loop = jnp.arange(N, dtype=jnp.int32)
    pad = EPAD - ETOT
    ppos = jnp.arange(pad, dtype=jnp.int32) % N
    src = jnp.concatenate([edge_index[0].astype(jnp.int32), loop, ppos])
    dst = jnp.concatenate([edge_index[1].astype(jnp.int32), loop, ppos])
    ew = jnp.concatenate([edge_weight, jnp.ones((N,), jnp.float32),
                          jnp.zeros((pad,), jnp.float32)])

    ewi = lax.bitcast_convert_type(ew, jnp.int32)
    edata0 = jnp.stack([src, dst, ewi], axis=0)
    edata0 = edata0.reshape(3, NW, NCHUNK, K).transpose(1, 2, 0, 3)
    edata = _sc_pre()(edata0.reshape(NW, NCHUNK * 3, K))
    edata = edata.reshape(NW, NCHUNK, 3, K)

    xw1 = _tc_mm(x, W1)
    acc1 = _sc_agg(H)(xw1, edata)
    xw2 = _tc_bn_mm(acc1, g1, be1, W2)
    acc2 = _sc_agg(H)(xw2, edata)
    W3p = jnp.pad(W3, ((0, 0), (0, CP - C)))
    xw3 = _tc_bn_mm(acc2, g2, be2, W3p)
    acc3 = _sc_agg(CP)(xw3, edata)
    b3p = jnp.pad(b3, (0, CP - C))
    return _tc_final(acc3, b3p)


# speedup vs baseline: 1.0123x; 1.0123x over previous
import functools

import jax
import jax.numpy as jnp
from jax import lax
from jax.experimental import pallas as pl
from jax.experimental.pallas import tpu as pltpu
from jax.experimental.pallas import tpu_sc as plsc

N = 10000
E = 320000
D = 128
H = 128
C = 40
CP = 128

NC = 2
NS = 16
L = 16
NW = NC * NS

NPAD = 10240
RPT = NPAD // NS
ETOT = E + N
K = 112
NCHUNK = 96
EW = NCHUNK * K
EPAD = EW * NW

@functools.cache
def _mesh():
    return plsc.VectorSubcoreMesh(
        core_axis_name="c", subcore_axis_name="s",
        num_cores=NC, num_subcores=NS)


_SC_PARAMS = pltpu.CompilerParams(needs_layout_passes=False)


def _sc_pre_body(ed_in, ed_out, acc, sbuf, tbuf, dinvv, ebw,
                 degsh, dinvsh):
    cid = lax.axis_index("c")
    sid = lax.axis_index("s")
    wid = cid * NS + sid

    zero = jnp.zeros((L,), jnp.float32)

    @pl.loop(0, NPAD // L)
    def _zero(i):
        acc[pl.ds(i * L, L)] = zero

    for woff in (0, NS):
        pltpu.sync_copy(ed_in.at[sid + woff], ebw)

        @pl.loop(0, NCHUNK)
        def _chunk(t):
            for j in range(K // L):
                d16 = ebw[t * 3 + 1, pl.ds(j * L, L)]
                w16 = plsc.bitcast(ebw[t * 3 + 2, pl.ds(j * L, L)],
                                   jnp.float32)
                plsc.addupdate_scatter(acc, [d16], w16)

    for s in range(NS):
        pltpu.sync_copy(acc.at[pl.ds(s * RPT, RPT)], degsh.at[s, sid])
    plsc.subcore_barrier()

    pltpu.sync_copy(degsh.at[sid], tbuf)

    @pl.loop(0, RPT // L)
    def _red(i):
        s = tbuf[0, pl.ds(i * L, L)]
        for w in range(1, NS):
            s = s + tbuf[w, pl.ds(i * L, L)]
        x = jnp.maximum(s, 1.0)
        yi = jnp.int32(0x5F3759DF) - (plsc.bitcast(x, jnp.int32) >> 1)
        y = plsc.bitcast(yi, jnp.float32)
        h = 0.5 * x
        y = y * (1.5 - h * y * y)
        y = y * (1.5 - h * y * y)
        y = y * (1.5 - h * y * y)
        sbuf[pl.ds(i * L, L)] = y

    pltpu.sync_copy(sbuf, dinvsh.at[sid])
    plsc.subcore_barrier()
    for s in range(NS):
        pltpu.sync_copy(dinvsh.at[s], dinvv.at[pl.ds(s * RPT, RPT)])

    pltpu.sync_copy(ed_in.at[wid], ebw)

    @pl.loop(0, NCHUNK)
    def _cchunk(t):
        for j in range(K // L):
            s16 = ebw[t * 3, pl.ds(j * L, L)]
            d16 = ebw[t * 3 + 1, pl.ds(j * L, L)]
            w16 = plsc.bitcast(ebw[t * 3 + 2, pl.ds(j * L, L)], jnp.float32)
            c16 = w16 * plsc.load_gather(dinvv, [s16]) \
                      * plsc.load_gather(dinvv, [d16])
            ebw[t * 3 + 2, pl.ds(j * L, L)] = plsc.bitcast(c16, jnp.int32)

    pltpu.sync_copy(ebw, ed_out.at[wid])


@functools.cache
def _sc_pre():
    return pl.kernel(
        _sc_pre_body,
        out_type=jax.ShapeDtypeStruct((NW, NCHUNK * 3, K), jnp.int32),
        mesh=_mesh(),
        compiler_params=_SC_PARAMS,
        scratch_types=[
            pltpu.VMEM((NPAD,), jnp.float32),
            pltpu.VMEM((RPT,), jnp.float32),
            pltpu.VMEM((NS, RPT), jnp.float32),
            pltpu.VMEM((NPAD,), jnp.float32),
            pltpu.VMEM((NCHUNK * 3, K), jnp.int32),
            pltpu.VMEM_SHARED((NS, NS, RPT), jnp.float32),
            pltpu.VMEM_SHARED((NS, RPT), jnp.float32),
        ],
    )


def _make_sc_agg(hp):
    def body(xw_hbm, ed_hbm, out_hbm,
             accs, eb0, eb1, eb2, eb3, eb4, eb5, rw0, rw1, rw2,
             es0, es1, es2, es3, es4, es5, gs0, gs1, gs2, ss0, ss1, ss2):
        cid = lax.axis_index("c")
        sid = lax.axis_index("s")
        wid = cid * NS + sid

        eb = (eb0, eb1, eb2, eb3, eb4, eb5)
        es = (es0, es1, es2, es3, es4, es5)
        rw = (rw0, rw1, rw2)
        gs = (gs0, gs1, gs2)
        ss = (ss0, ss1, ss2)

        pltpu.sync_copy(ed_hbm.at[wid, 0], eb0)
        pltpu.sync_copy(ed_hbm.at[wid, 1], eb1)
        pltpu.async_copy(ed_hbm.at[wid, 2], eb2, es2)
        pltpu.async_copy(xw_hbm.at[eb0.at[0]], rw0, gs0)
        pltpu.async_copy(xw_hbm.at[eb1.at[0]], rw1, gs1)

        zero = jnp.zeros((L,), jnp.float32)

        @pl.loop(0, K)
        def _z0(r):
            for j in range(hp // L):
                rw2[r, pl.ds(j * L, L)] = zero

        nfull = RPT // K
        rem = RPT - nfull * K
        for i in range(nfull):
            pltpu.sync_copy(rw2, accs.at[pl.ds(sid * RPT + i * K, K)])
        if rem:
            pltpu.sync_copy(rw2.at[pl.ds(0, rem)],
                            accs.at[pl.ds(sid * RPT + nfull * K, rem)])

        plsc.subcore_barrier()

        @pl.loop(0, NCHUNK // 6)
        def _grp(g):
            for b in range(6):
                u = g * 6 + b
                b3 = b % 3
                bp3 = (b + 2) % 3
                bn3 = (b + 2) % 3

                @pl.when(u + 3 < NCHUNK)
                def _():
                    pltpu.async_copy(ed_hbm.at[wid, u + 3],
                                     eb[(b + 3) % 6], es[(b + 3) % 6])

                pltpu.make_async_copy(
                    xw_hbm.at[eb[b].at[0]], rw[b3], gs[b3]).wait()

                cref = eb[b].at[2]

                @pl.loop(0, K, unroll=4)
                def _row(r):
                    cr = plsc.bitcast(
                        plsc.load_gather(cref, [jnp.full((L,), r, jnp.int32)]),
                        jnp.float32)
                    for j in range(hp // L):
                        rw[b3][r, pl.ds(j * L, L)] = \
                            rw[b3][r, pl.ds(j * L, L)] * cr

                @pl.when(u >= 1)
                def _():
                    pltpu.make_async_copy(
                        rw[bp3], accs.at[eb[(b + 5) % 6].at[1]],
                        ss[bp3]).wait()

                pltpu.async_copy(rw[b3], accs.at[eb[b].at[1]], ss[b3],
                                 add=True)

                @pl.when(u + 2 < NCHUNK)
                def _():
                    pltpu.make_async_copy(ed_hbm.at[wid, u + 2],
                                          eb[(b + 2) % 6],
                                          es[(b + 2) % 6]).wait()
                    pltpu.async_copy(xw_hbm.at[eb[(b + 2) % 6].at[0]],
                                     rw[bn3], gs[bn3])

        ul = NCHUNK - 1
        pltpu.make_async_copy(rw[ul % 3], accs.at[eb[ul % 6].at[1]],
                              ss[ul % 3]).wait()
        plsc.subcore_barrier()
        pltpu.sync_copy(accs.at[pl.ds(sid * RPT, RPT)],
                        out_hbm.at[cid, pl.ds(sid * RPT, RPT)])

    return pl.kernel(
        body,
        out_type=jax.ShapeDtypeStruct((NC, NPAD, hp), jnp.float32),
        mesh=_mesh(),
        compiler_params=_SC_PARAMS,
        scratch_types=[
            pltpu.VMEM_SHARED((NPAD, hp), jnp.float32),
            pltpu.VMEM((3, K), jnp.int32),
            pltpu.VMEM((3, K), jnp.int32),
            pltpu.VMEM((3, K), jnp.int32),
            pltpu.VMEM((3, K), jnp.int32),
            pltpu.VMEM((3, K), jnp.int32),
            pltpu.VMEM((3, K), jnp.int32),
            pltpu.VMEM((K, hp), jnp.float32),
            pltpu.VMEM((K, hp), jnp.float32),
            pltpu.VMEM((K, hp), jnp.float32),
            pltpu.SemaphoreType.DMA,
            pltpu.SemaphoreType.DMA,
            pltpu.SemaphoreType.DMA,
            pltpu.SemaphoreType.DMA,
            pltpu.SemaphoreType.DMA,
            pltpu.SemaphoreType.DMA,
            pltpu.SemaphoreType.DMA,
            pltpu.SemaphoreType.DMA,
            pltpu.SemaphoreType.DMA,
            pltpu.SemaphoreType.DMA,
            pltpu.SemaphoreType.DMA,
            pltpu.SemaphoreType.DMA,
        ],
    )


_sc_agg = functools.cache(_make_sc_agg)


def _tc_mm_body(x_ref, w_ref, o_ref):
    o_ref[...] = jnp.dot(x_ref[...], w_ref[...],
                         preferred_element_type=jnp.float32)


def _tc_mm(x, w):
    return pl.pallas_call(
        _tc_mm_body,
        out_shape=jax.ShapeDtypeStruct((x.shape[0], w.shape[1]), jnp.float32),
    )(x, w)


def _tc_bn_mm_body(acc_ref, g_ref, be_ref, w_ref, o_ref):
    a = acc_ref[0] + acc_ref[1]
    mean = jnp.sum(a, axis=0, keepdims=True) * (1.0 / N)
    dev = a - mean
    rmask = lax.broadcasted_iota(jnp.int32, (NPAD, 1), 0) < N
    devm = jnp.where(rmask, dev, 0.0)
    var = jnp.sum(devm * devm, axis=0, keepdims=True) * (1.0 / N)
    h = dev * lax.rsqrt(var + 1e-5) * g_ref[...] + be_ref[...]
    h = jnp.maximum(h, 0.0)
    o_ref[...] = jnp.dot(h, w_ref[...], preferred_element_type=jnp.float32)


def _tc_bn_mm(accs, g, be, w):
    return pl.pallas_call(
        _tc_bn_mm_body,
        out_shape=jax.ShapeDtypeStruct((NPAD, w.shape[1]), jnp.float32),
    )(accs, g.reshape(1, -1), be.reshape(1, -1), w)


def _tc_final_body(acc_ref, b_ref, o_ref):
    z = acc_ref[0] + acc_ref[1] + b_ref[...]
    cmask = lax.broadcasted_iota(jnp.int32, (1, CP), 1) < C
    z = jnp.where(cmask, z, -1e30)
    m = jnp.max(z, axis=1, keepdims=True)
    e = jnp.where(cmask, jnp.exp(z - m), 0.0)
    s = jnp.sum(e, axis=1, keepdims=True)
    out = z - m - jnp.log(s)
    o_ref[...] = out[:N, :C]


def _tc_final(accs, b3p):
    return pl.pallas_call(
        _tc_final_body,
        out_shape=jax.ShapeDtypeStruct((N, C), jnp.float32),
    )(accs, b3p.reshape(1, -1))


def kernel(x, edge_index, edge_weight, W1, b1, g1, be1,
           W2, b2, g2, be2, W3, b3):
    loop = jnp.arange(N, dtype=jnp.int32)
    pad = EPAD - ETOT
    ppos = jnp.arange(pad, dtype=jnp.int32) % N
    src = jnp.concatenate([edge_index[0].astype(jnp.int32), loop, ppos])
    dst = jnp.concatenate([edge_index[1].astype(jnp.int32), loop, ppos])
    ew = jnp.concatenate([edge_weight, jnp.ones((N,), jnp.float32),
                          jnp.zeros((pad,), jnp.float32)])

    ewi = lax.bitcast_convert_type(ew, jnp.int32)
    edata0 = jnp.stack([src, dst, ewi], axis=0)
    edata0 = edata0.reshape(3, NW, NCHUNK, K).transpose(1, 2, 0, 3)
    edata = _sc_pre()(edata0.reshape(NW, NCHUNK * 3, K))
    edata = edata.reshape(NW, NCHUNK, 3, K)

    xw1 = _tc_mm(x, W1)
    acc1 = _sc_agg(H)(xw1, edata)
    xw2 = _tc_bn_mm(acc1, g1, be1, W2)
    acc2 = _sc_agg(H)(xw2, edata)
    W3p = jnp.pad(W3, ((0, 0), (0, CP - C)))
    xw3 = _tc_bn_mm(acc2, g2, be2, W3p)
    acc3 = _sc_agg(CP)(xw3, edata)
    b3p = jnp.pad(b3, (0, CP - C))
    return _tc_final(acc3, b3p)
